# baseline (speedup 1.0000x reference)
import functools

import jax
import jax.numpy as jnp
from jax import lax
from jax.experimental import pallas as pl
from jax.experimental.pallas import tpu as pltpu
from jax.experimental.pallas import tpu_sc as plsc

_VOCAB = 1000000
_DIM = 32
_B = 16384
_NEG = 20

_info = plsc.get_sparse_core_info()
_NC, _NS, _L = _info.num_cores, _info.num_subcores, _info.num_lanes
_NW = _NC * _NS
_BPW = _B // _NW
_CB = 128
_NCHUNK = _BPW // _CB
_CE = _CB * _NEG
_NSPLIT = _CE // _CB


def _sc_scores(u_w, v_w, logp, pos_u, pos_v, neg_flat):
    mesh = plsc.VectorSubcoreMesh(core_axis_name="c", subcore_axis_name="s")

    @functools.partial(
        pl.kernel,
        mesh=mesh,
        compiler_params=pltpu.CompilerParams(
            needs_layout_passes=False, use_tc_tiling_on_sc=False),
        out_type=[
            jax.ShapeDtypeStruct((_B,), jnp.float32),
            jax.ShapeDtypeStruct((_B * _NEG,), jnp.float32),
        ],
        scratch_types=[
            pltpu.VMEM((_CB,), jnp.int32),
            pltpu.VMEM((_CB,), jnp.int32),
            pltpu.VMEM((_CE,), jnp.int32),
            pltpu.VMEM((_CB, _DIM), jnp.float32),
            pltpu.VMEM((_CB, _DIM), jnp.float32),
            pltpu.VMEM((_CE, _DIM), jnp.float32),
            pltpu.VMEM((_CB,), jnp.float32),
            pltpu.VMEM((_CB,), jnp.float32),
            pltpu.VMEM((_CE,), jnp.float32),
            pltpu.SemaphoreType.DMA,
        ],
    )
    def sc_k(u_hbm, v_hbm, lp_hbm, pu_hbm, pv_hbm, nv_hbm,
             pos_out, neg_out,
             idx_u, idx_v, idx_n, u_rows, v_rows, n_rows, lp_v,
             pos_st, neg_st, sem):
        wid = lax.axis_index("s") * _NC + lax.axis_index("c")

        def chunk_body(ci, carry):
            base = wid * _BPW + ci * _CB
            pltpu.sync_copy(pu_hbm.at[pl.ds(base, _CB)], idx_u)
            pltpu.sync_copy(pv_hbm.at[pl.ds(base, _CB)], idx_v)
            pltpu.sync_copy(nv_hbm.at[pl.ds(base * _NEG, _CE)], idx_n)
            cps = [
                pltpu.async_copy(u_hbm.at[idx_u], u_rows, sem),
                pltpu.async_copy(v_hbm.at[idx_v], v_rows, sem),
                pltpu.async_copy(lp_hbm.at[idx_u], lp_v, sem),
            ]
            for j in range(_NSPLIT):
                cps.append(pltpu.async_copy(
                    v_hbm.at[idx_n.at[pl.ds(j * _CB, _CB)]],
                    n_rows.at[pl.ds(j * _CB, _CB)], sem))
            for cp in cps:
                cp.wait()

            lane = lax.iota(jnp.int32, _L)

            def group_body(g, carry2):
                bvec = g * _L + lane
                us = [plsc.load_gather(u_rows,
                                       [bvec, jnp.full((_L,), d, jnp.int32)])
                      for d in range(_DIM)]
                quad_h = us[0] * us[0]
                for d in range(1, _DIM):
                    quad_h = quad_h + us[d] * us[d]
                quad_h = 0.5 * quad_h
                linacc = jnp.zeros((_L,), jnp.float32)
                for d in range(_DIM):
                    vv = plsc.load_gather(v_rows,
                                          [bvec, jnp.full((_L,), d, jnp.int32)])
                    linacc = linacc + us[d] * vv
                lpv = lp_v[pl.ds(g * _L, _L)]
                bias = lpv - quad_h
                pos_st[pl.ds(g * _L, _L)] = linacc + bias

                def n_body(n, carry3):
                    evec = bvec * _NEG + n
                    acc = jnp.zeros((_L,), jnp.float32)
                    for d in range(_DIM):
                        nv = plsc.load_gather(
                            n_rows, [evec, jnp.full((_L,), d, jnp.int32)])
                        acc = acc + nv * us[d]
                    plsc.store_scatter(neg_st, [evec], acc + bias)
                    return carry3

                lax.fori_loop(0, _NEG, n_body, 0)
                return carry2

            lax.fori_loop(0, _CB // _L, group_body, 0)
            pltpu.sync_copy(pos_st, pos_out.at[pl.ds(base, _CB)])
            pltpu.sync_copy(neg_st, neg_out.at[pl.ds(base * _NEG, _CE)])
            return carry

        lax.fori_loop(0, _NCHUNK, chunk_body, 0)

    return sc_k(u_w, v_w, logp, pos_u, pos_v, neg_flat)


def _tc_reduce(pos_raw, neg_raw):
    def body(pos_ref, neg_ref, out_ref):
        p = jnp.clip(pos_ref[...], -10.0, 10.0)
        q = jnp.clip(neg_ref[...], -10.0, 10.0)
        tot = jnp.sum(jnp.log1p(jnp.exp(-p))) + jnp.sum(jnp.log1p(jnp.exp(q)))
        out_ref[0, 0] = tot * (1.0 / _B)

    return pl.pallas_call(
        body,
        out_shape=jax.ShapeDtypeStruct((1, 1), jnp.float32),
        out_specs=pl.BlockSpec(memory_space=pltpu.SMEM),
    )(pos_raw, neg_raw)


def kernel(u_weight, v_weight, log_priors, pos_u, pos_v, neg_v):
    neg_flat = neg_v.reshape(-1).astype(jnp.int32)
    pos_raw, neg_raw = _sc_scores(
        u_weight, v_weight, log_priors,
        pos_u.astype(jnp.int32), pos_v.astype(jnp.int32), neg_flat)
    out = _tc_reduce(pos_raw.reshape(_B // 128, 128),
                     neg_raw.reshape(_B * _NEG // 128, 128))
    return out.reshape(())
